# baseline (speedup 1.0000x reference)
import functools

import jax
import jax.numpy as jnp
from jax import lax
from jax.experimental import pallas as pl
from jax.experimental.pallas import tpu as pltpu
from jax.experimental.pallas import tpu_sc as plsc

N = 10000
E = 320000
ND = 128
ED = 16
GD = 128
HD = 128

NC = 2
NS = 16
NW = NC * NS
NHALF = 1
EH = E // NHALF
NCH = 125
SLAB = 624
TAIL = N - NS * SLAB

_EPS = 1e-5


def _ln(x, g, b):
    m = jnp.mean(x, axis=-1, keepdims=True)
    d = x - m
    v = jnp.mean(d * d, axis=-1, keepdims=True)
    return d * lax.rsqrt(v + _EPS) * g + b


def _node_pre_body(node_blk, ep_blk, gf, ln_nr_g, ln_nr_b, ln_nc_g, ln_nc_b,
                   ln_n_g, ln_n_b, ln_g_g, ln_g_b, nr_w, nr_b, nc_w,
                   nf_w, nf_b, gf_w, avg16, g16t, b16t,
                   nrp_out, nc_out, nf_out, z_out):
    x = node_blk[...]
    g0 = jnp.dot(jnp.maximum(_ln(gf[...], ln_g_g[...], ln_g_b[...]), 0.0),
                 gf_w[...], preferred_element_type=jnp.float32)
    nr = jnp.maximum(_ln(x, ln_nr_g[...], ln_nr_b[...]), 0.0)
    nrp_out[...] = (jnp.dot(nr, nr_w[...], preferred_element_type=jnp.float32)
                    + nr_b[...] + g0)
    ncv = jnp.maximum(_ln(x, ln_nc_g[...], ln_nc_b[...]), 0.0)
    nc_out[...] = jnp.dot(ncv, nc_w[...], preferred_element_type=jnp.float32)
    nfv = jnp.maximum(_ln(x, ln_n_g[...], ln_n_b[...]), 0.0)
    nf_out[...] = (jnp.dot(nfv, nf_w[...], preferred_element_type=jnp.float32)
                   + nf_b[...])
    ep = ep_blk[...]
    m = jnp.dot(ep, avg16[...], preferred_element_type=jnp.float32)
    sq = jnp.dot(ep * ep, avg16[...], preferred_element_type=jnp.float32)
    v = sq - m * m
    z = (ep - m) * lax.rsqrt(v + _EPS) * g16t[...] + b16t[...]
    z_out[...] = jnp.maximum(z, 0.0)


def _node_pre(node_feats, ep, graph_feats, p):
    BT = 1000
    EPB = (E // 8) // (N // BT)
    grid = N // BT
    row_spec = pl.BlockSpec((BT, ND), lambda i: (i, 0))
    ep_spec = pl.BlockSpec((EPB, 128), lambda i: (i, 0))
    full = lambda a: pl.BlockSpec(a.shape, lambda i: tuple(0 for _ in a.shape))
    avg16 = jnp.kron(jnp.eye(8, dtype=jnp.float32),
                     jnp.full((ED, ED), 1.0 / ED, jnp.float32))
    args = (graph_feats,
            p['e_ln_nr_g'].reshape(1, ND), p['e_ln_nr_b'].reshape(1, ND),
            p['e_ln_nc_g'].reshape(1, ND), p['e_ln_nc_b'].reshape(1, ND),
            p['n_ln_n_g'].reshape(1, ND), p['n_ln_n_b'].reshape(1, ND),
            p['e_ln_g_g'].reshape(1, GD), p['e_ln_g_b'].reshape(1, GD),
            p['e_nr_w'], p['e_nr_b'].reshape(1, HD), p['e_nc_w'],
            p['n_nf_w'], p['n_nf_b'].reshape(1, HD), p['e_gf_w'],
            avg16,
            jnp.tile(p['e_ln_e_g'], 8).reshape(1, 128),
            jnp.tile(p['e_ln_e_b'], 8).reshape(1, 128))
    return pl.pallas_call(
        _node_pre_body,
        grid=(grid,),
        in_specs=[row_spec, ep_spec] + [full(a) for a in args],
        out_specs=[row_spec, row_spec, row_spec, ep_spec],
        out_shape=[jax.ShapeDtypeStruct((N, HD), jnp.float32)] * 3
                  + [jax.ShapeDtypeStruct((E // 8, 128), jnp.float32)],
    )(node_feats, ep, *args)


def _make_sc_gather(ne):
    epw = ne // NW
    ch = epw // NCH

    def body(nrp_hbm, nc_hbm, rowi_hbm, coli_hbm, out_hbm,
             idxr, idxc, bufa0, bufb0, bufa1, bufb1,
             sga0, sgb0, sga1, sgb1, sout0, sout1):
        c = lax.axis_index("c")
        s = lax.axis_index("s")
        wid = s * NC + c
        pltpu.sync_copy(rowi_hbm.at[wid], idxr)
        pltpu.sync_copy(coli_hbm.at[wid], idxc)
        base = wid * epw

        bufa = (bufa0, bufa1)
        bufb = (bufb0, bufb1)
        sga = (sga0, sga1)
        sgb = (sgb0, sgb1)
        sout = (sout0, sout1)

        def gather_desc(j, sl):
            return (pltpu.make_async_copy(nrp_hbm.at[idxr.at[j]], bufa[sl],
                                          sga[sl]),
                    pltpu.make_async_copy(nc_hbm.at[idxc.at[j]], bufb[sl],
                                          sgb[sl]))

        def start_gather(j, sl):
            da, db = gather_desc(j, sl)
            da.start()
            db.start()

        def wait_gather(j, sl):
            da, db = gather_desc(j, sl)
            da.wait()
            db.wait()

        def add(sl):
            a, b = bufa[sl], bufb[sl]

            def add_row(r, carry2):
                for l in range(HD // 16):
                    slc = pl.ds(l * 16, 16)
                    a[r, slc] = a[r, slc] + b[r, slc]
                return carry2

            lax.fori_loop(0, ch, add_row, 0)

        def out_desc(j, sl):
            return pltpu.make_async_copy(
                bufa[sl], out_hbm.at[pl.ds(base + j * ch, ch)], sout[sl])

        start_gather(0, 0)
        start_gather(1, 1)
        wait_gather(0, 0)
        add(0)
        out_desc(0, 0).start()

        def outer(k, carry):
            for b in (0, 1):
                j = 1 + 2 * k + b
                sl = (1 + b) % 2
                nx = (sl + 1) % 2
                out_desc(j - 1, nx).wait()
                start_gather(j + 1, nx)
                wait_gather(j, sl)
                add(sl)
                out_desc(j, sl).start()
            return carry

        lax.fori_loop(0, (NCH - 3) // 2, outer, 0)

        out_desc(NCH - 3, 0).wait()
        start_gather(NCH - 1, 0)
        wait_gather(NCH - 2, 1)
        add(1)
        out_desc(NCH - 2, 1).start()
        wait_gather(NCH - 1, 0)
        add(0)
        out_desc(NCH - 2, 1).wait()
        out_desc(NCH - 1, 0).start()
        out_desc(NCH - 1, 0).wait()

    mesh = plsc.VectorSubcoreMesh(core_axis_name="c", subcore_axis_name="s")
    return functools.partial(
        pl.kernel,
        out_type=jax.ShapeDtypeStruct((ne, HD), jnp.float32),
        mesh=mesh,
        scratch_types=[
            pltpu.VMEM((NCH, ch), jnp.int32),
            pltpu.VMEM((NCH, ch), jnp.int32),
            pltpu.VMEM((ch, HD), jnp.float32),
            pltpu.VMEM((ch, HD), jnp.float32),
            pltpu.VMEM((ch, HD), jnp.float32),
            pltpu.VMEM((ch, HD), jnp.float32),
            pltpu.SemaphoreType.DMA,
            pltpu.SemaphoreType.DMA,
            pltpu.SemaphoreType.DMA,
            pltpu.SemaphoreType.DMA,
            pltpu.SemaphoreType.DMA,
            pltpu.SemaphoreType.DMA,
        ],
    )(body)


def _edge_mlp_body(gath_blk, ef_blk, z_blk, ef_w, ln_h_g, ln_h_b,
                   fc1_w, fc1_b, out_blk):
    h = jnp.dot(z_blk[...], ef_w[...], preferred_element_type=jnp.float32)
    carry = gath_blk[...] + h
    carry = jnp.maximum(_ln(carry, ln_h_g[...], ln_h_b[...]), 0.0)
    carry = jnp.dot(carry, fc1_w[...], preferred_element_type=jnp.float32)
    out_blk[...] = ef_blk[...] + carry + fc1_b[...]


def _edge_mlp(gathered, edge_feats, z, p):
    ne = gathered.shape[0]
    TE = 2000 if ne % 2560 else 2560
    grid = ne // TE
    full = lambda a: pl.BlockSpec(a.shape, lambda i: tuple(0 for _ in a.shape))
    args = (p['e_ef_w'], p['e_ln_h_g'].reshape(1, HD),
            p['e_ln_h_b'].reshape(1, HD), p['e_fc1_w'],
            p['e_fc1_b'].reshape(1, ED))
    return pl.pallas_call(
        _edge_mlp_body,
        grid=(grid,),
        in_specs=[pl.BlockSpec((TE, HD), lambda i: (i, 0)),
                  pl.BlockSpec((TE, ED), lambda i: (i, 0)),
                  pl.BlockSpec((TE, ED), lambda i: (i, 0))]
                 + [full(a) for a in args],
        out_specs=pl.BlockSpec((TE, ED), lambda i: (i, 0)),
        out_shape=jax.ShapeDtypeStruct((ne, ED), jnp.float32),
    )(gathered, edge_feats, z, *args)


def _make_sc_scatter(ne):
    epw = ne // NW
    ch = epw // NCH

    def body(eo_hbm, rowi_hbm, zeros_hbm, out_hbm,
             i0, i1, i2, i3, v16_0, v16_1, v128_0, v128_1, acc,
             sv0, sv1, ss0, ss1):
        c = lax.axis_index("c")
        s = lax.axis_index("s")
        wid = s * NC + c
        pltpu.sync_copy(zeros_hbm.at[pl.ds(s * SLAB, SLAB)],
                        acc.at[pl.ds(s * SLAB, SLAB)])

        @pl.when(s == NS - 1)
        def _():
            pltpu.sync_copy(zeros_hbm.at[pl.ds(NS * SLAB, TAIL)],
                            acc.at[pl.ds(NS * SLAB, TAIL)])

        pltpu.sync_copy(zeros_hbm.at[pl.ds(0, ch)], v128_0)
        pltpu.sync_copy(zeros_hbm.at[pl.ds(0, ch)], v128_1)
        plsc.subcore_barrier()
        base = wid * epw

        idx = (i0, i1, i2, i3)
        v16 = (v16_0, v16_1)
        v128 = (v128_0, v128_1)
        sv = (sv0, sv1)
        ss = (ss0, ss1)

        def load_descs(j, jm4):
            sl = jm4 % 2
            return (pltpu.make_async_copy(
                        eo_hbm.at[pl.ds(base + j * ch, ch)], v16[sl], sv[sl]),
                    pltpu.make_async_copy(
                        rowi_hbm.at[pl.ds(base + j * ch, ch)], idx[jm4],
                        sv[sl]))

        def load_start(j, jm4):
            de, di = load_descs(j, jm4)
            de.start()
            di.start()

        def load_wait(j, jm4):
            de, di = load_descs(j, jm4)
            de.wait()
            di.wait()

        def scat_desc(j, jm4):
            return pltpu.make_async_copy(v128[jm4 % 2], acc.at[idx[jm4]],
                                         ss[jm4 % 2])

        def expand(sl):
            a, b = v16[sl], v128[sl]

            def expand_row(e, carry2):
                b[e, pl.ds(0, ED)] = a[e, :]
                return carry2

            lax.fori_loop(0, ch, expand_row, 0)

        load_start(0, 0)
        load_start(1, 1)
        load_wait(0, 0)
        expand(0)
        scat_desc(0, 0).start(add=True)
        load_start(2, 2)
        load_wait(1, 1)
        expand(1)
        scat_desc(1, 1).start(add=True)

        def outer(k, carry):
            for b in (0, 1, 2, 3):
                j = 2 + 4 * k + b
                jm4 = (2 + b) % 4
                sl = jm4 % 2
                load_start(j + 1, (jm4 + 1) % 4)
                load_wait(j, jm4)
                scat_desc(j - 2, (jm4 + 2) % 4).wait()
                expand(sl)
                scat_desc(j, jm4).start(add=True)
            return carry

        lax.fori_loop(0, (NCH - 5) // 4, outer, 0)

        load_start(NCH - 2, 3)
        load_wait(NCH - 3, 2)
        scat_desc(NCH - 5, 0).wait()
        expand(0)
        scat_desc(NCH - 3, 2).start(add=True)

        load_start(NCH - 1, 0)
        load_wait(NCH - 2, 3)
        scat_desc(NCH - 4, 1).wait()
        expand(1)
        scat_desc(NCH - 2, 3).start(add=True)

        load_wait(NCH - 1, 0)
        scat_desc(NCH - 3, 2).wait()
        expand(0)
        scat_desc(NCH - 1, 0).start(add=True)
        scat_desc(NCH - 2, 3).wait()
        scat_desc(NCH - 1, 0).wait()

        plsc.subcore_barrier()
        pltpu.sync_copy(acc.at[pl.ds(s * SLAB, SLAB)],
                        out_hbm.at[pl.ds(c * N + s * SLAB, SLAB)])

        @pl.when(s == NS - 1)
        def _():
            pltpu.sync_copy(acc.at[pl.ds(NS * SLAB, TAIL)],
                            out_hbm.at[pl.ds(c * N + NS * SLAB, TAIL)])

    mesh = plsc.VectorSubcoreMesh(core_axis_name="c", subcore_axis_name="s")
    return functools.partial(
        pl.kernel,
        out_type=jax.ShapeDtypeStruct((NC * N, HD), jnp.float32),
        mesh=mesh,
        scratch_types=[
            pltpu.VMEM((ch,), jnp.int32),
            pltpu.VMEM((ch,), jnp.int32),
            pltpu.VMEM((ch,), jnp.int32),
            pltpu.VMEM((ch,), jnp.int32),
            pltpu.VMEM((ch, ED), jnp.float32),
            pltpu.VMEM((ch, ED), jnp.float32),
            pltpu.VMEM((ch, HD), jnp.float32),
            pltpu.VMEM((ch, HD), jnp.float32),
            pltpu.VMEM_SHARED((N, HD), jnp.float32),
            pltpu.SemaphoreType.DMA,
            pltpu.SemaphoreType.DMA,
            pltpu.SemaphoreType.DMA,
            pltpu.SemaphoreType.DMA,
        ],
    )(body)


def _node_upd_body(*refs):
    nparts = 2 * NHALF
    parts = refs[:nparts]
    (node_blk, nf_blk, gfeats, ln_e_g, ln_e_b, ef_w, ln_h_g, ln_h_b,
     fc1_w, fc1_b, node_out, graph_out) = refs[nparts:]
    i = pl.program_id(0)
    msgs = parts[0][...]
    for pr in parts[1:]:
        msgs = msgs + pr[...]
    m = jnp.maximum(_ln(msgs, ln_e_g[...], ln_e_b[...]), 0.0)
    m = jnp.dot(m, ef_w[...], preferred_element_type=jnp.float32)
    h = jnp.maximum(_ln(nf_blk[...] + m, ln_h_g[...], ln_h_b[...]), 0.0)
    out = jnp.dot(h, fc1_w[...], preferred_element_type=jnp.float32) + fc1_b[...]
    node_out[...] = node_blk[...] + out[:, :ND]
    gp = jnp.sum(out[:, ND:], axis=0, keepdims=True)

    @pl.when(i == 0)
    def _():
        graph_out[...] = gfeats[...] + gp

    @pl.when(i > 0)
    def _():
        graph_out[...] = graph_out[...] + gp


def _node_upd(parts, node_feats, nf, graph_feats, p):
    BT = 1000
    grid = N // BT
    full = lambda a: pl.BlockSpec(a.shape, lambda i: tuple(0 for _ in a.shape))
    row16 = pl.BlockSpec((BT, ED), lambda i: (i, 0))
    row128 = pl.BlockSpec((BT, ND), lambda i: (i, 0))
    args = (graph_feats,
            p['n_ln_e_g'].reshape(1, ED), p['n_ln_e_b'].reshape(1, ED),
            p['n_ef_w'], p['n_ln_h_g'].reshape(1, HD),
            p['n_ln_h_b'].reshape(1, HD), p['n_fc1_w'],
            p['n_fc1_b'].reshape(1, ND + GD))
    return pl.pallas_call(
        _node_upd_body,
        grid=(grid,),
        in_specs=[row16] * len(parts) + [row128, row128]
                 + [full(a) for a in args],
        out_specs=[row128, pl.BlockSpec((1, GD), lambda i: (0, 0))],
        out_shape=[jax.ShapeDtypeStruct((N, ND), jnp.float32),
                   jax.ShapeDtypeStruct((1, GD), jnp.float32)],
    )(*parts, node_feats, nf, *args)


def kernel(node_feats, edge_feats, edge_index, graph_feats, graph_index, params):
    del graph_index
    row = edge_index[0]
    col = edge_index[1]

    ep = edge_feats.reshape(E // 8, 128)
    nrp, nc, nf, zp = _node_pre(node_feats, ep, graph_feats, params)
    z = zp.reshape(E, ED)
    zeros = jnp.zeros((N, HD), jnp.float32)

    epw = EH // NW
    ch = epw // NCH
    gatherer = _make_sc_gather(EH)
    scatterer = _make_sc_scatter(EH)

    eo_halves = []
    parts = []
    for h in range(NHALF):
        rh = lax.slice_in_dim(row, h * EH, (h + 1) * EH)
        colh = lax.slice_in_dim(col, h * EH, (h + 1) * EH)
        efh = lax.slice_in_dim(edge_feats, h * EH, (h + 1) * EH)
        zh = lax.slice_in_dim(z, h * EH, (h + 1) * EH)
        row_r = rh.reshape(NW, NCH, ch)
        col_r = colh.reshape(NW, NCH, ch)
        gath = gatherer(nrp, nc, row_r, col_r)
        eo = _edge_mlp(gath, efh, zh, params)
        msgs_p = scatterer(eo, rh, zeros)
        eo_halves.append(eo)
        parts.append(msgs_p[:N, :ED])
        parts.append(msgs_p[N:, :ED])

    edge_out = jnp.concatenate(eo_halves, axis=0)
    node_out, graph_out = _node_upd(parts, node_feats, nf, graph_feats, params)
    return (node_out, edge_out, graph_out)


# speedup vs baseline: 1.2561x; 1.2561x over previous
import functools

import jax
import jax.numpy as jnp
from jax import lax
from jax.experimental import pallas as pl
from jax.experimental.pallas import tpu as pltpu
from jax.experimental.pallas import tpu_sc as plsc

N = 10000
E = 320000
ND = 128
ED = 16
GD = 128
HD = 128

NC = 2
NS = 16
NW = NC * NS
NHALF = 1
EH = E // NHALF
NCH = 125
SLAB = 624
TAIL = N - NS * SLAB

_EPS = 1e-5


def _ln(x, g, b):
    m = jnp.mean(x, axis=-1, keepdims=True)
    d = x - m
    v = jnp.mean(d * d, axis=-1, keepdims=True)
    return d * lax.rsqrt(v + _EPS) * g + b


def _node_pre_body(node_blk, gf, ln_nr_g, ln_nr_b, ln_nc_g, ln_nc_b,
                   ln_n_g, ln_n_b, ln_g_g, ln_g_b, nr_w, nr_b, nc_w,
                   nf_w, nf_b, gf_w, nrp_out, nc_out, nf_out):
    x = node_blk[...]
    g0 = jnp.dot(jnp.maximum(_ln(gf[...], ln_g_g[...], ln_g_b[...]), 0.0),
                 gf_w[...], preferred_element_type=jnp.float32)
    nr = jnp.maximum(_ln(x, ln_nr_g[...], ln_nr_b[...]), 0.0)
    nrp_out[...] = (jnp.dot(nr, nr_w[...], preferred_element_type=jnp.float32)
                    + nr_b[...] + g0)
    ncv = jnp.maximum(_ln(x, ln_nc_g[...], ln_nc_b[...]), 0.0)
    nc_out[...] = jnp.dot(ncv, nc_w[...], preferred_element_type=jnp.float32)
    nfv = jnp.maximum(_ln(x, ln_n_g[...], ln_n_b[...]), 0.0)
    nf_out[...] = (jnp.dot(nfv, nf_w[...], preferred_element_type=jnp.float32)
                   + nf_b[...])


def _node_pre(node_feats, graph_feats, p):
    BT = 1000
    grid = N // BT
    row_spec = pl.BlockSpec((BT, ND), lambda i: (i, 0))
    full = lambda a: pl.BlockSpec(a.shape, lambda i: tuple(0 for _ in a.shape))
    args = (graph_feats,
            p['e_ln_nr_g'].reshape(1, ND), p['e_ln_nr_b'].reshape(1, ND),
            p['e_ln_nc_g'].reshape(1, ND), p['e_ln_nc_b'].reshape(1, ND),
            p['n_ln_n_g'].reshape(1, ND), p['n_ln_n_b'].reshape(1, ND),
            p['e_ln_g_g'].reshape(1, GD), p['e_ln_g_b'].reshape(1, GD),
            p['e_nr_w'], p['e_nr_b'].reshape(1, HD), p['e_nc_w'],
            p['n_nf_w'], p['n_nf_b'].reshape(1, HD), p['e_gf_w'])
    return pl.pallas_call(
        _node_pre_body,
        grid=(grid,),
        in_specs=[row_spec] + [full(a) for a in args],
        out_specs=[row_spec, row_spec, row_spec],
        out_shape=[jax.ShapeDtypeStruct((N, HD), jnp.float32)] * 3,
    )(node_feats, *args)


def _make_sc_gather(ne):
    epw = ne // NW
    ch = epw // NCH

    def body(nrp_hbm, nc_hbm, rowi_hbm, coli_hbm, out_hbm,
             idxr, idxc, bufa0, bufb0, bufa1, bufb1,
             sga0, sgb0, sga1, sgb1, sout0, sout1):
        c = lax.axis_index("c")
        s = lax.axis_index("s")
        wid = s * NC + c
        pltpu.sync_copy(rowi_hbm.at[wid], idxr)
        pltpu.sync_copy(coli_hbm.at[wid], idxc)
        base = wid * epw

        bufa = (bufa0, bufa1)
        bufb = (bufb0, bufb1)
        sga = (sga0, sga1)
        sgb = (sgb0, sgb1)
        sout = (sout0, sout1)

        def gather_desc(j, sl):
            return (pltpu.make_async_copy(nrp_hbm.at[idxr.at[j]], bufa[sl],
                                          sga[sl]),
                    pltpu.make_async_copy(nc_hbm.at[idxc.at[j]], bufb[sl],
                                          sgb[sl]))

        def start_gather(j, sl):
            da, db = gather_desc(j, sl)
            da.start()
            db.start()

        def wait_gather(j, sl):
            da, db = gather_desc(j, sl)
            da.wait()
            db.wait()

        def add(sl):
            a, b = bufa[sl], bufb[sl]

            def add_row(r, carry2):
                for l in range(HD // 16):
                    slc = pl.ds(l * 16, 16)
                    a[r, slc] = a[r, slc] + b[r, slc]
                return carry2

            lax.fori_loop(0, ch, add_row, 0)

        def out_desc(j, sl):
            return pltpu.make_async_copy(
                bufa[sl], out_hbm.at[pl.ds(base + j * ch, ch)], sout[sl])

        start_gather(0, 0)
        start_gather(1, 1)
        wait_gather(0, 0)
        add(0)
        out_desc(0, 0).start()

        def outer(k, carry):
            for b in (0, 1):
                j = 1 + 2 * k + b
                sl = (1 + b) % 2
                nx = (sl + 1) % 2
                out_desc(j - 1, nx).wait()
                start_gather(j + 1, nx)
                wait_gather(j, sl)
                add(sl)
                out_desc(j, sl).start()
            return carry

        lax.fori_loop(0, (NCH - 3) // 2, outer, 0)

        out_desc(NCH - 3, 0).wait()
        start_gather(NCH - 1, 0)
        wait_gather(NCH - 2, 1)
        add(1)
        out_desc(NCH - 2, 1).start()
        wait_gather(NCH - 1, 0)
        add(0)
        out_desc(NCH - 2, 1).wait()
        out_desc(NCH - 1, 0).start()
        out_desc(NCH - 1, 0).wait()

    mesh = plsc.VectorSubcoreMesh(core_axis_name="c", subcore_axis_name="s")
    return functools.partial(
        pl.kernel,
        out_type=jax.ShapeDtypeStruct((ne, HD), jnp.float32),
        mesh=mesh,
        scratch_types=[
            pltpu.VMEM((NCH, ch), jnp.int32),
            pltpu.VMEM((NCH, ch), jnp.int32),
            pltpu.VMEM((ch, HD), jnp.float32),
            pltpu.VMEM((ch, HD), jnp.float32),
            pltpu.VMEM((ch, HD), jnp.float32),
            pltpu.VMEM((ch, HD), jnp.float32),
            pltpu.SemaphoreType.DMA,
            pltpu.SemaphoreType.DMA,
            pltpu.SemaphoreType.DMA,
            pltpu.SemaphoreType.DMA,
            pltpu.SemaphoreType.DMA,
            pltpu.SemaphoreType.DMA,
        ],
    )(body)


def _edge_mlp_body(gath_blk, ef_blk, ln_e_g, ln_e_b, ef_w, ln_h_g, ln_h_b,
                   fc1_w, fc1_b, out_blk):
    e = ef_blk[...]
    h = jnp.maximum(_ln(e, ln_e_g[...], ln_e_b[...]), 0.0)
    h = jnp.dot(h, ef_w[...], preferred_element_type=jnp.float32)
    carry = gath_blk[...] + h
    carry = jnp.maximum(_ln(carry, ln_h_g[...], ln_h_b[...]), 0.0)
    carry = jnp.dot(carry, fc1_w[...], preferred_element_type=jnp.float32)
    out_blk[...] = e + carry + fc1_b[...]


def _edge_mlp(gathered, edge_feats, p):
    ne = gathered.shape[0]
    TE = 2000 if ne % 2560 else 2560
    grid = ne // TE
    full = lambda a: pl.BlockSpec(a.shape, lambda i: tuple(0 for _ in a.shape))
    args = (p['e_ln_e_g'].reshape(1, ED), p['e_ln_e_b'].reshape(1, ED),
            p['e_ef_w'], p['e_ln_h_g'].reshape(1, HD),
            p['e_ln_h_b'].reshape(1, HD), p['e_fc1_w'],
            p['e_fc1_b'].reshape(1, ED))
    return pl.pallas_call(
        _edge_mlp_body,
        grid=(grid,),
        in_specs=[pl.BlockSpec((TE, HD), lambda i: (i, 0)),
                  pl.BlockSpec((TE, ED), lambda i: (i, 0))]
                 + [full(a) for a in args],
        out_specs=pl.BlockSpec((TE, ED), lambda i: (i, 0)),
        out_shape=jax.ShapeDtypeStruct((ne, ED), jnp.float32),
    )(gathered, edge_feats, *args)


def _make_sc_scatter(ne):
    epw = ne // NW
    ch = epw // NCH

    def body(eo_hbm, rowi_hbm, zeros_hbm, out_hbm,
             i0, i1, i2, i3, v16_0, v16_1, v128_0, v128_1, acc,
             sv0, sv1, ss0, ss1):
        c = lax.axis_index("c")
        s = lax.axis_index("s")
        wid = s * NC + c
        pltpu.sync_copy(zeros_hbm.at[pl.ds(s * SLAB, SLAB)],
                        acc.at[pl.ds(s * SLAB, SLAB)])

        @pl.when(s == NS - 1)
        def _():
            pltpu.sync_copy(zeros_hbm.at[pl.ds(NS * SLAB, TAIL)],
                            acc.at[pl.ds(NS * SLAB, TAIL)])

        pltpu.sync_copy(zeros_hbm.at[pl.ds(0, ch)], v128_0)
        pltpu.sync_copy(zeros_hbm.at[pl.ds(0, ch)], v128_1)
        plsc.subcore_barrier()
        base = wid * epw

        idx = (i0, i1, i2, i3)
        v16 = (v16_0, v16_1)
        v128 = (v128_0, v128_1)
        sv = (sv0, sv1)
        ss = (ss0, ss1)

        def load_descs(j, jm4):
            sl = jm4 % 2
            return (pltpu.make_async_copy(
                        eo_hbm.at[pl.ds(base + j * ch, ch)], v16[sl], sv[sl]),
                    pltpu.make_async_copy(
                        rowi_hbm.at[pl.ds(base + j * ch, ch)], idx[jm4],
                        sv[sl]))

        def load_start(j, jm4):
            de, di = load_descs(j, jm4)
            de.start()
            di.start()

        def load_wait(j, jm4):
            de, di = load_descs(j, jm4)
            de.wait()
            di.wait()

        def scat_desc(j, jm4):
            return pltpu.make_async_copy(v128[jm4 % 2], acc.at[idx[jm4]],
                                         ss[jm4 % 2])

        def expand(sl):
            a, b = v16[sl], v128[sl]

            def expand_row(e, carry2):
                b[e, pl.ds(0, ED)] = a[e, :]
                return carry2

            lax.fori_loop(0, ch, expand_row, 0)

        load_start(0, 0)
        load_start(1, 1)
        load_wait(0, 0)
        expand(0)
        scat_desc(0, 0).start(add=True)
        load_start(2, 2)
        load_wait(1, 1)
        expand(1)
        scat_desc(1, 1).start(add=True)

        def outer(k, carry):
            for b in (0, 1, 2, 3):
                j = 2 + 4 * k + b
                jm4 = (2 + b) % 4
                sl = jm4 % 2
                load_start(j + 1, (jm4 + 1) % 4)
                load_wait(j, jm4)
                scat_desc(j - 2, (jm4 + 2) % 4).wait()
                expand(sl)
                scat_desc(j, jm4).start(add=True)
            return carry

        lax.fori_loop(0, (NCH - 5) // 4, outer, 0)

        load_start(NCH - 2, 3)
        load_wait(NCH - 3, 2)
        scat_desc(NCH - 5, 0).wait()
        expand(0)
        scat_desc(NCH - 3, 2).start(add=True)

        load_start(NCH - 1, 0)
        load_wait(NCH - 2, 3)
        scat_desc(NCH - 4, 1).wait()
        expand(1)
        scat_desc(NCH - 2, 3).start(add=True)

        load_wait(NCH - 1, 0)
        scat_desc(NCH - 3, 2).wait()
        expand(0)
        scat_desc(NCH - 1, 0).start(add=True)
        scat_desc(NCH - 2, 3).wait()
        scat_desc(NCH - 1, 0).wait()

        plsc.subcore_barrier()
        pltpu.sync_copy(acc.at[pl.ds(s * SLAB, SLAB)],
                        out_hbm.at[pl.ds(c * N + s * SLAB, SLAB)])

        @pl.when(s == NS - 1)
        def _():
            pltpu.sync_copy(acc.at[pl.ds(NS * SLAB, TAIL)],
                            out_hbm.at[pl.ds(c * N + NS * SLAB, TAIL)])

    mesh = plsc.VectorSubcoreMesh(core_axis_name="c", subcore_axis_name="s")
    return functools.partial(
        pl.kernel,
        out_type=jax.ShapeDtypeStruct((NC * N, HD), jnp.float32),
        mesh=mesh,
        scratch_types=[
            pltpu.VMEM((ch,), jnp.int32),
            pltpu.VMEM((ch,), jnp.int32),
            pltpu.VMEM((ch,), jnp.int32),
            pltpu.VMEM((ch,), jnp.int32),
            pltpu.VMEM((ch, ED), jnp.float32),
            pltpu.VMEM((ch, ED), jnp.float32),
            pltpu.VMEM((ch, HD), jnp.float32),
            pltpu.VMEM((ch, HD), jnp.float32),
            pltpu.VMEM_SHARED((N, HD), jnp.float32),
            pltpu.SemaphoreType.DMA,
            pltpu.SemaphoreType.DMA,
            pltpu.SemaphoreType.DMA,
            pltpu.SemaphoreType.DMA,
        ],
    )(body)


def _node_upd_body(*refs):
    nparts = 2 * NHALF
    parts = refs[:nparts]
    (node_blk, nf_blk, gfeats, ln_e_g, ln_e_b, ef_w, ln_h_g, ln_h_b,
     fc1_w, fc1_b, node_out, graph_out) = refs[nparts:]
    i = pl.program_id(0)
    msgs = parts[0][...]
    for pr in parts[1:]:
        msgs = msgs + pr[...]
    m = jnp.maximum(_ln(msgs, ln_e_g[...], ln_e_b[...]), 0.0)
    m = jnp.dot(m, ef_w[...], preferred_element_type=jnp.float32)
    h = jnp.maximum(_ln(nf_blk[...] + m, ln_h_g[...], ln_h_b[...]), 0.0)
    out = jnp.dot(h, fc1_w[...], preferred_element_type=jnp.float32) + fc1_b[...]
    node_out[...] = node_blk[...] + out[:, :ND]
    gp = jnp.sum(out[:, ND:], axis=0, keepdims=True)

    @pl.when(i == 0)
    def _():
        graph_out[...] = gfeats[...] + gp

    @pl.when(i > 0)
    def _():
        graph_out[...] = graph_out[...] + gp


def _node_upd(parts, node_feats, nf, graph_feats, p):
    BT = 1000
    grid = N // BT
    full = lambda a: pl.BlockSpec(a.shape, lambda i: tuple(0 for _ in a.shape))
    row16 = pl.BlockSpec((BT, ED), lambda i: (i, 0))
    row128 = pl.BlockSpec((BT, ND), lambda i: (i, 0))
    args = (graph_feats,
            p['n_ln_e_g'].reshape(1, ED), p['n_ln_e_b'].reshape(1, ED),
            p['n_ef_w'], p['n_ln_h_g'].reshape(1, HD),
            p['n_ln_h_b'].reshape(1, HD), p['n_fc1_w'],
            p['n_fc1_b'].reshape(1, ND + GD))
    return pl.pallas_call(
        _node_upd_body,
        grid=(grid,),
        in_specs=[row16] * len(parts) + [row128, row128]
                 + [full(a) for a in args],
        out_specs=[row128, pl.BlockSpec((1, GD), lambda i: (0, 0))],
        out_shape=[jax.ShapeDtypeStruct((N, ND), jnp.float32),
                   jax.ShapeDtypeStruct((1, GD), jnp.float32)],
    )(*parts, node_feats, nf, *args)


def kernel(node_feats, edge_feats, edge_index, graph_feats, graph_index, params):
    del graph_index
    row = edge_index[0]
    col = edge_index[1]

    nrp, nc, nf = _node_pre(node_feats, graph_feats, params)
    zeros = jnp.zeros((N, HD), jnp.float32)

    epw = EH // NW
    ch = epw // NCH
    gatherer = _make_sc_gather(EH)
    scatterer = _make_sc_scatter(EH)

    eo_halves = []
    parts = []
    for h in range(NHALF):
        rh = lax.slice_in_dim(row, h * EH, (h + 1) * EH)
        colh = lax.slice_in_dim(col, h * EH, (h + 1) * EH)
        efh = lax.slice_in_dim(edge_feats, h * EH, (h + 1) * EH)
        row_r = rh.reshape(NW, NCH, ch)
        col_r = colh.reshape(NW, NCH, ch)
        gath = gatherer(nrp, nc, row_r, col_r)
        eo = _edge_mlp(gath, efh, params)
        msgs_p = scatterer(eo, rh, zeros)
        eo_halves.append(eo)
        parts.append(msgs_p[:N, :ED])
        parts.append(msgs_p[N:, :ED])

    edge_out = jnp.concatenate(eo_halves, axis=0)
    node_out, graph_out = _node_upd(parts, node_feats, nf, graph_feats, params)
    return (node_out, edge_out, graph_out)
